# baseline (speedup 1.0000x reference)
import functools

import jax
import jax.numpy as jnp
from jax import lax
from jax.experimental import pallas as pl
from jax.experimental.pallas import tpu as pltpu
from jax.experimental.pallas import tpu_sc as plsc

N_WORDS = 1000000
D = 64
BATCH = 16384
HIST = 50
B = BATCH * HIST

NC = 2
NS = 16
NW = NC * NS
NBT = BATCH // 128
NUNIT = HIST * NBT
UPW = NUNIT // NW
NBUF = 4

_mesh = plsc.VectorSubcoreMesh(
    core_axis_name="c", subcore_axis_name="s", num_cores=NC, num_subcores=NS
)


@functools.partial(
    pl.kernel,
    out_type=jax.ShapeDtypeStruct((HIST, 8, NBT, 1024), jnp.float32),
    mesh=_mesh,
    compiler_params=pltpu.CompilerParams(
        use_tc_tiling_on_sc=False, needs_layout_passes=False
    ),
    scratch_types=[
        pltpu.VMEM((UPW, 128), jnp.int32),
        pltpu.VMEM((NBUF, 128, D), jnp.float32),
        pltpu.VMEM((NBUF, 8 * 1024), jnp.float32),
    ]
    + [pltpu.SemaphoreType.DMA] * (2 * NBUF),
)
def _emb_lookup(idx_hbm, table_hbm, out_hbm, idx_v, rows_v, tbuf, *sems):
    gsem = sems[:NBUF]
    osem = sems[NBUF:]
    wid = lax.axis_index("s") * NC + lax.axis_index("c")
    pltpu.sync_copy(idx_hbm.at[wid], idx_v)

    def gather(j, b):
        return pltpu.make_async_copy(
            table_hbm.at[idx_v.at[j]], rows_v.at[b], gsem[b]
        )

    def writeback_descs(j, b):
        u = wid * UPW + j
        h = u // NBT
        bt = u % NBT
        return [
            pltpu.make_async_copy(
                tbuf.at[b, pl.ds(dt * 1024, 1024)],
                out_hbm.at[h, dt, bt],
                osem[b],
            )
            for dt in range(8)
        ]

    base_ids = [lax.iota(jnp.int32, 16) * 128 + 2048 * k for k in range(4)]

    def transpose(b):
        rv = rows_v.at[b]
        tb = tbuf.at[b]

        @plsc.parallel_loop(0, 128, unroll=8)
        def _row(r):
            off = jnp.full((16,), r, jnp.int32)
            for k in range(4):
                v = rv[r, pl.ds(k * 16, 16)]
                plsc.store_scatter(tb, [base_ids[k] + off], v)

    for b in range(NBUF):
        gather(b, b).start()

    @pl.loop(0, UPW, step=NBUF)
    def _group(j0):
        for b in range(NBUF):
            j = j0 + b
            gather(j, b).wait()

            @pl.when(j >= NBUF)
            def _():
                for d in writeback_descs(j - NBUF, b):
                    d.wait()

            transpose(b)
            for d in writeback_descs(j, b):
                d.start()

            @pl.when(j + NBUF < UPW)
            def _():
                gather(j + NBUF, b).start()

    for b in range(NBUF):
        for d in writeback_descs(UPW - NBUF + b, b):
            d.wait()


def kernel(x, x_0):
    idx = x.T.reshape(NW, UPW, 128)
    out4 = _emb_lookup(idx, x_0)
    out5 = out4.reshape(HIST, 8, NBT, 8, 128)
    return out5.transpose(2, 4, 0, 1, 3).reshape(BATCH, HIST, D)


# speedup vs baseline: 1.7377x; 1.7377x over previous
import functools

import jax
import jax.numpy as jnp
from jax import lax
from jax.experimental import pallas as pl
from jax.experimental.pallas import tpu as pltpu
from jax.experimental.pallas import tpu_sc as plsc

N_WORDS = 1000000
D = 64
BATCH = 16384
HIST = 50
B = BATCH * HIST

NC = 2
NS = 16
NW = NC * NS
NBT = BATCH // 128
NUNIT = HIST * NBT
UPW = NUNIT // NW
NBUF = 4

_mesh = plsc.VectorSubcoreMesh(
    core_axis_name="c", subcore_axis_name="s", num_cores=NC, num_subcores=NS
)


@functools.partial(
    pl.kernel,
    out_type=jax.ShapeDtypeStruct((HIST, 8, NBT, 1024), jnp.float32),
    mesh=_mesh,
    compiler_params=pltpu.CompilerParams(
        use_tc_tiling_on_sc=False, needs_layout_passes=False
    ),
    scratch_types=[
        pltpu.VMEM((UPW, 128), jnp.int32),
        pltpu.VMEM((NBUF, 128, D), jnp.float32),
        pltpu.VMEM((NBUF, 8 * 1024), jnp.float32),
    ]
    + [pltpu.SemaphoreType.DMA] * (2 * NBUF),
)
def _emb_lookup(idx_hbm, table_hbm, out_hbm, idx_v, rows_v, tbuf, *sems):
    gsem = sems[:NBUF]
    osem = sems[NBUF:]
    wid = lax.axis_index("s") * NC + lax.axis_index("c")
    pltpu.sync_copy(idx_hbm.at[wid], idx_v)

    def gather(j, b):
        return pltpu.make_async_copy(
            table_hbm.at[idx_v.at[j]], rows_v.at[b], gsem[b]
        )

    def writeback_descs(j, b):
        u = wid * UPW + j
        h = u // NBT
        bt = u % NBT
        return [
            pltpu.make_async_copy(
                tbuf.at[b, pl.ds(dt * 1024, 1024)],
                out_hbm.at[h, dt, bt],
                osem[b],
            )
            for dt in range(8)
        ]

    lane = lax.iota(jnp.int32, 16)
    col_ids = [16 * k + lane for k in range(4)]
    sbase = [2048 * k + 128 * lane for k in range(4)]

    def transpose(b):
        rv = rows_v.at[b]
        tb = tbuf.at[b]

        @plsc.parallel_loop(0, 128, unroll=8)
        def _diag(r0):
            row = (jnp.full((16,), r0, jnp.int32) + lane) & 127
            for k in range(4):
                v = plsc.load_gather(rv, [row, col_ids[k]])
                plsc.store_scatter(tb, [sbase[k] + row], v)

    for b in range(NBUF):
        gather(b, b).start()

    @pl.loop(0, UPW, step=NBUF)
    def _group(j0):
        for b in range(NBUF):
            j = j0 + b
            gather(j, b).wait()

            @pl.when(j >= NBUF)
            def _():
                for d in writeback_descs(j - NBUF, b):
                    d.wait()

            transpose(b)
            for d in writeback_descs(j, b):
                d.start()

            @pl.when(j + NBUF < UPW)
            def _():
                gather(j + NBUF, b).start()

    for b in range(NBUF):
        for d in writeback_descs(UPW - NBUF + b, b):
            d.wait()


def kernel(x, x_0):
    idx = x.T.reshape(NW, UPW, 128)
    out4 = _emb_lookup(idx, x_0)
    out5 = out4.reshape(HIST, 8, NBT, 8, 128)
    return out5.transpose(2, 4, 0, 1, 3).reshape(BATCH, HIST, D)
